# baseline (speedup 1.0000x reference)
import functools

import jax
import jax.numpy as jnp
from jax import lax
from jax.experimental import pallas as pl
from jax.experimental.pallas import tpu as pltpu
from jax.experimental.pallas import tpu_sc as plsc

_BC = 256


def kernel(x, table):
    b, l = x.shape
    v, d = table.shape
    dp = 128
    table_p = jnp.pad(table, ((0, 0), (0, dp - d)))
    xt_flat = x.T.reshape(b * l)
    info = plsc.get_sparse_core_info()
    nw = info.num_cores * info.num_subcores
    b_per_w = b // nw
    halves = b_per_w // _BC
    units = l * halves
    mesh = plsc.VectorSubcoreMesh(core_axis_name="c", subcore_axis_name="s")

    @functools.partial(
        pl.kernel,
        out_type=jax.ShapeDtypeStruct((l, d, b), jnp.float32),
        mesh=mesh,
        scratch_types=(
            [pltpu.VMEM((_BC,), jnp.int32) for _ in range(2)]
            + [pltpu.VMEM((_BC, dp), jnp.float32) for _ in range(2)]
            + [pltpu.VMEM((d, _BC), jnp.float32) for _ in range(2)]
            + [pltpu.SemaphoreType.DMA((2,)),
               pltpu.SemaphoreType.DMA((2,)),
               pltpu.SemaphoreType.DMA((2,))]
        ),
        compiler_params=pltpu.CompilerParams(
            use_tc_tiling_on_sc=True, needs_layout_passes=False),
    )
    def emb(x_hbm, table_hbm, out_hbm, *refs):
        idx_v = refs[0:2]
        rows_v = refs[2:4]
        trans_v = refs[4:6]
        idx_sem, gat_sem, out_sem = refs[6:9]
        wid = lax.axis_index("s") * info.num_cores + lax.axis_index("c")
        b0 = wid * b_per_w

        def unit_off(u):
            ll = lax.div(u, halves)
            h = lax.rem(u, halves)
            return ll * b + b0 + h * _BC, ll, h

        for p in range(2):
            off, _, _ = unit_off(jnp.int32(p))
            pltpu.async_copy(x_hbm.at[pl.ds(off, _BC)], idx_v[p],
                             idx_sem.at[p])
        pltpu.make_async_copy(x_hbm.at[pl.ds(0, _BC)], idx_v[0],
                              idx_sem.at[0]).wait()
        pltpu.async_copy(table_hbm.at[idx_v[0]], rows_v[0], gat_sem.at[0])

        lane = lax.iota(jnp.int32, 16)
        bb_bases = [lane + 16 * g for g in range(_BC // 16)]

        @pl.loop(0, units)
        def _unit(u):
            p = lax.rem(u, 2)
            _, ll, h = unit_off(u)

            pltpu.make_async_copy(
                table_hbm.at[idx_v[0]], rows_v[0], gat_sem.at[p]).wait()

            @pl.when(u + 1 < units)
            def _():
                pltpu.make_async_copy(
                    x_hbm.at[pl.ds(0, _BC)], idx_v[0],
                    idx_sem.at[1 - p]).wait()
                for q in range(2):
                    @pl.when(1 - p == q)
                    def _():
                        pltpu.async_copy(table_hbm.at[idx_v[q]], rows_v[q],
                                         gat_sem.at[1 - p])

            @pl.when(u + 2 < units)
            def _():
                off2, _, _ = unit_off(u + 2)
                for q in range(2):
                    @pl.when(p == q)
                    def _():
                        pltpu.async_copy(x_hbm.at[pl.ds(off2, _BC)],
                                         idx_v[q], idx_sem.at[p])

            @pl.when(u >= 2)
            def _():
                pltpu.make_async_copy(
                    trans_v[0], out_hbm.at[0, :, pl.ds(0, _BC)],
                    out_sem.at[p]).wait()

            for q in range(2):
                @pl.when(p == q)
                def _():
                    @pl.loop(0, d)
                    def _d(dd):
                        col = jnp.full((16,), dd, jnp.int32)
                        for g in range(_BC // 16):
                            vals = plsc.load_gather(
                                rows_v[q], [bb_bases[g], col])
                            trans_v[q][dd, pl.ds(16 * g, 16)] = vals

            for q in range(2):
                @pl.when(p == q)
                def _():
                    pltpu.async_copy(
                        trans_v[q],
                        out_hbm.at[ll, :, pl.ds(b0 + h * _BC, _BC)],
                        out_sem.at[p])

        for p in range(2):
            pltpu.make_async_copy(
                trans_v[0], out_hbm.at[0, :, pl.ds(0, _BC)],
                out_sem.at[p]).wait()

    out_t = emb(xt_flat, table_p)
    return jnp.transpose(out_t, (2, 0, 1))


# speedup vs baseline: 2.2627x; 2.2627x over previous
import functools

import jax
import jax.numpy as jnp
from jax import lax
from jax.experimental import pallas as pl
from jax.experimental.pallas import tpu as pltpu
from jax.experimental.pallas import tpu_sc as plsc

_NBUF = 4


def kernel(x, table):
    b, l = x.shape
    v, d = table.shape
    n = b * l
    dp = 128
    table_p = jnp.pad(table, ((0, 0), (0, dp - d)))
    x_flat = x.reshape(n)
    info = plsc.get_sparse_core_info()
    nw = info.num_cores * info.num_subcores
    rows_per_w = b // nw
    groups = rows_per_w // _NBUF
    mesh = plsc.VectorSubcoreMesh(core_axis_name="c", subcore_axis_name="s")

    @functools.partial(
        pl.kernel,
        out_type=jax.ShapeDtypeStruct((b, l, dp), jnp.float32),
        mesh=mesh,
        scratch_types=(
            [pltpu.VMEM((l,), jnp.int32) for _ in range(_NBUF)]
            + [pltpu.VMEM((l, dp), jnp.float32) for _ in range(_NBUF)]
            + [pltpu.SemaphoreType.DMA((_NBUF,)),
               pltpu.SemaphoreType.DMA((_NBUF,)),
               pltpu.SemaphoreType.DMA((_NBUF,))]
        ),
        compiler_params=pltpu.CompilerParams(use_tc_tiling_on_sc=True),
    )
    def emb(x_hbm, table_hbm, out_hbm, *refs):
        idx_v = refs[:_NBUF]
        rows_v = refs[_NBUF:2 * _NBUF]
        idx_sem, gat_sem, out_sem = refs[2 * _NBUF:]
        wid = lax.axis_index("s") * info.num_cores + lax.axis_index("c")
        base = wid * rows_per_w

        for bb in range(_NBUF):
            pltpu.async_copy(
                x_hbm.at[pl.ds((base + bb) * l, l)], idx_v[bb],
                idx_sem.at[bb])

        @pl.loop(0, groups)
        def _grp(g):
            s0 = base + g * _NBUF
            descs = []
            for bb in range(_NBUF):
                @pl.when(g > 0)
                def _():
                    pltpu.make_async_copy(
                        rows_v[bb], out_hbm.at[0], out_sem.at[bb]).wait()

                pltpu.make_async_copy(
                    x_hbm.at[pl.ds(0, l)], idx_v[bb], idx_sem.at[bb]).wait()

                descs.append(pltpu.async_copy(
                    table_hbm.at[idx_v[bb]], rows_v[bb], gat_sem.at[bb]))

            for bb in range(_NBUF):
                descs[bb].wait()
                pltpu.async_copy(rows_v[bb], out_hbm.at[s0 + bb],
                                 out_sem.at[bb])

                @pl.when(g + 1 < groups)
                def _():
                    pltpu.async_copy(
                        x_hbm.at[pl.ds((s0 + bb + _NBUF) * l, l)],
                        idx_v[bb], idx_sem.at[bb])

        for bb in range(_NBUF):
            pltpu.make_async_copy(
                rows_v[bb], out_hbm.at[0], out_sem.at[bb]).wait()

    out_full = emb(x_flat, table_p)
    return out_full[:, :, :d]


# speedup vs baseline: 2.8815x; 1.2735x over previous
import functools

import jax
import jax.numpy as jnp
from jax import lax
from jax.experimental import pallas as pl
from jax.experimental.pallas import tpu as pltpu
from jax.experimental.pallas import tpu_sc as plsc

_NBUF = 8


def kernel(x, table):
    b, l = x.shape
    v, d = table.shape
    n = b * l
    dp = 128
    x_flat = x.reshape(n)
    info = plsc.get_sparse_core_info()
    nw = info.num_cores * info.num_subcores
    rows_per_w = b // nw
    groups = rows_per_w // _NBUF
    mesh = plsc.VectorSubcoreMesh(core_axis_name="c", subcore_axis_name="s")

    @functools.partial(
        pl.kernel,
        out_type=jax.ShapeDtypeStruct((b, l, dp), jnp.float32),
        mesh=mesh,
        scratch_types=(
            [pltpu.VMEM((l,), jnp.int32) for _ in range(_NBUF)]
            + [pltpu.VMEM((l, d), jnp.float32) for _ in range(_NBUF)]
            + [pltpu.SemaphoreType.DMA((_NBUF,)),
               pltpu.SemaphoreType.DMA((_NBUF,)),
               pltpu.SemaphoreType.DMA((_NBUF,))]
        ),
        compiler_params=pltpu.CompilerParams(use_tc_tiling_on_sc=False),
    )
    def emb(x_hbm, table_hbm, out_hbm, *refs):
        idx_v = refs[:_NBUF]
        rows_v = refs[_NBUF:2 * _NBUF]
        idx_sem, gat_sem, out_sem = refs[2 * _NBUF:]
        wid = lax.axis_index("s") * info.num_cores + lax.axis_index("c")
        base = wid * rows_per_w

        for bb in range(_NBUF):
            pltpu.async_copy(
                x_hbm.at[pl.ds((base + bb) * l, l)], idx_v[bb],
                idx_sem.at[bb])

        @pl.loop(0, groups)
        def _grp(g):
            s0 = base + g * _NBUF
            descs = []
            for bb in range(_NBUF):
                @pl.when(g > 0)
                def _():
                    pltpu.make_async_copy(
                        rows_v[bb], out_hbm.at[0, :, pl.ds(0, d)],
                        out_sem.at[bb]).wait()

                pltpu.make_async_copy(
                    x_hbm.at[pl.ds(0, l)], idx_v[bb], idx_sem.at[bb]).wait()

                descs.append(pltpu.async_copy(
                    table_hbm.at[idx_v[bb]], rows_v[bb], gat_sem.at[bb]))

            for bb in range(_NBUF):
                descs[bb].wait()
                pltpu.async_copy(rows_v[bb],
                                 out_hbm.at[s0 + bb, :, pl.ds(0, d)],
                                 out_sem.at[bb])

                @pl.when(g + 1 < groups)
                def _():
                    pltpu.async_copy(
                        x_hbm.at[pl.ds((s0 + bb + _NBUF) * l, l)],
                        idx_v[bb], idx_sem.at[bb])

        for bb in range(_NBUF):
            pltpu.make_async_copy(
                rows_v[bb], out_hbm.at[0, :, pl.ds(0, d)],
                out_sem.at[bb]).wait()

    out_full = emb(x_flat, table)
    return out_full[:, :, :d]
